# baseline (speedup 1.0000x reference)
import functools

import jax
import jax.numpy as jnp
from jax.experimental import pallas as pl
from jax.experimental.pallas import tpu as pltpu
from jax.experimental.pallas import tpu_sc as plsc

FEATURE_SIZE = 128
FP_LENGTH = 512
RADIUS = 3
N_NODES = 10000
N_EDGES = 320000

_BR = 1000

_NC, _NS, _L = 2, 16, 16
_NW = _NC * _NS
_CHUNK = 128
_NCH = 80
_EPW = _NCH * _CHUNK
_EPAD = _NW * _EPW
_BCH = 16
_NB = _NCH // _BCH
_NP = 10240
_RPT = _NP // _NS
_ZC = 128
_ZCH = _RPT // _ZC


def _sc_agg_body(x_hbm, src_hbm, dst_hbm, out_hbm,
                 sidx0, didx0, sidx1, didx1, rows_a, rows_b, agg_sh,
                 sem_a, sem_b, sem_i):
    c = jax.lax.axis_index("c")
    s = jax.lax.axis_index("s")

    zero = jnp.zeros((_L,), jnp.float32)
    qpr = FEATURE_SIZE // _L

    def _zrow(i, carry):
        rows_a[i // qpr, pl.ds((i % qpr) * _L, _L)] = zero
        return carry

    jax.lax.fori_loop(0, _ZC * qpr, _zrow, 0)

    def _zcopy(k, carry):
        pltpu.sync_copy(rows_a,
                        agg_sh.at[pl.ds(s * _RPT + k * _ZC, _ZC)])
        return carry

    jax.lax.fori_loop(0, _ZCH, _zcopy, 0)
    plsc.subcore_barrier()

    ibufs = [(sidx0, didx0), (sidx1, didx1)]
    pltpu.sync_copy(src_hbm.at[c, s, pl.ds(0, _BCH)], sidx0)
    pltpu.sync_copy(dst_hbm.at[c, s, pl.ds(0, _BCH)], didx0)
    for b in range(_NB):
        sidx, didx = ibufs[b % 2]
        if b + 1 < _NB:
            nsidx, ndidx = ibufs[(b + 1) % 2]
            pltpu.async_copy(src_hbm.at[c, s, pl.ds((b + 1) * _BCH, _BCH)],
                             nsidx, sem_i)
            pltpu.async_copy(dst_hbm.at[c, s, pl.ds((b + 1) * _BCH, _BCH)],
                             ndidx, sem_i)
        pltpu.async_copy(x_hbm.at[sidx.at[0]], rows_a, sem_a)

        def _pair(i, carry):
            j0 = i * 2
            pltpu.async_copy(x_hbm.at[sidx.at[j0 + 1]], rows_b, sem_b)
            pltpu.make_async_copy(x_hbm.at[sidx.at[j0]], rows_a,
                                  sem_a).wait()
            pltpu.sync_copy(rows_a, agg_sh.at[didx.at[j0]], add=True)

            @pl.when(j0 + 2 < _BCH)
            def _():
                pltpu.async_copy(x_hbm.at[sidx.at[j0 + 2]], rows_a, sem_a)

            pltpu.make_async_copy(x_hbm.at[sidx.at[j0 + 1]], rows_b,
                                  sem_b).wait()
            pltpu.sync_copy(rows_b, agg_sh.at[didx.at[j0 + 1]], add=True)
            return carry

        jax.lax.fori_loop(0, _BCH // 2, _pair, 0)
        if b + 1 < _NB:
            nsidx, ndidx = ibufs[(b + 1) % 2]
            pltpu.make_async_copy(
                src_hbm.at[c, s, pl.ds((b + 1) * _BCH, _BCH)], nsidx,
                sem_i).wait()
            pltpu.make_async_copy(
                dst_hbm.at[c, s, pl.ds((b + 1) * _BCH, _BCH)], ndidx,
                sem_i).wait()
    plsc.subcore_barrier()

    pltpu.sync_copy(agg_sh.at[pl.ds(s * _RPT, _RPT)],
                    out_hbm.at[c, pl.ds(s * _RPT, _RPT)])


_sc_agg = pl.kernel(
    _sc_agg_body,
    out_type=jax.ShapeDtypeStruct((_NC, _NP, FEATURE_SIZE), jnp.float32),
    mesh=plsc.VectorSubcoreMesh(core_axis_name="c", subcore_axis_name="s"),
    scratch_types=[
        pltpu.VMEM((_BCH, _CHUNK), jnp.int32),
        pltpu.VMEM((_BCH, _CHUNK), jnp.int32),
        pltpu.VMEM((_BCH, _CHUNK), jnp.int32),
        pltpu.VMEM((_BCH, _CHUNK), jnp.int32),
        pltpu.VMEM((_CHUNK, FEATURE_SIZE), jnp.float32),
        pltpu.VMEM((_CHUNK, FEATURE_SIZE), jnp.float32),
        pltpu.VMEM_SHARED((_NP, FEATURE_SIZE), jnp.float32),
        pltpu.SemaphoreType.DMA,
        pltpu.SemaphoreType.DMA,
        pltpu.SemaphoreType.DMA,
    ],
)


def _dense_body(feats_ref, agg_ref, w1_ref, b1_ref, w2_ref, b2_ref,
                h_ref, fp_ref):
    i = pl.program_id(0)
    ns = feats_ref[...] + agg_ref[0] + agg_ref[1]
    h = jax.lax.dot_general(ns, w1_ref[...], (((1,), (1,)), ((), ())),
                            preferred_element_type=jnp.float32)
    h = jnp.maximum(h + b1_ref[...], 0.0)
    h_ref[...] = h
    logits = jax.lax.dot_general(h, w2_ref[...], (((1,), (1,)), ((), ())),
                                 preferred_element_type=jnp.float32)
    logits = logits + b2_ref[...]
    m = jnp.max(logits, axis=1, keepdims=True)
    e = jnp.exp(logits - m)
    p = e / jnp.sum(e, axis=1, keepdims=True)
    part = jnp.sum(p, axis=0, keepdims=True)

    @pl.when(i == 0)
    def _():
        fp_ref[...] = part

    @pl.when(i != 0)
    def _():
        fp_ref[...] = fp_ref[...] + part


def _make_dense(interpret=False):
    grid = (N_NODES // _BR,)
    return pl.pallas_call(
        _dense_body,
        grid=grid,
        in_specs=[
            pl.BlockSpec((_BR, FEATURE_SIZE), lambda i: (i, 0)),
            pl.BlockSpec((_NC, _BR, FEATURE_SIZE), lambda i: (0, i, 0)),
            pl.BlockSpec((FEATURE_SIZE, FEATURE_SIZE), lambda i: (0, 0)),
            pl.BlockSpec((1, FEATURE_SIZE), lambda i: (0, 0)),
            pl.BlockSpec((FP_LENGTH, FEATURE_SIZE), lambda i: (0, 0)),
            pl.BlockSpec((1, FP_LENGTH), lambda i: (0, 0)),
        ],
        out_specs=[
            pl.BlockSpec((_BR, FEATURE_SIZE), lambda i: (i, 0)),
            pl.BlockSpec((1, FP_LENGTH), lambda i: (0, 0)),
        ],
        out_shape=[
            jax.ShapeDtypeStruct((N_NODES, FEATURE_SIZE), jnp.float32),
            jax.ShapeDtypeStruct((1, FP_LENGTH), jnp.float32),
        ],
        interpret=interpret,
    )


def kernel(x, edge_index, W1, b1, W2, b2, interpret=False):
    dense = _make_dense(interpret)
    npad = _EPAD - N_EDGES
    src = jnp.concatenate(
        [edge_index[0].astype(jnp.int32),
         jnp.zeros((npad,), jnp.int32)]).reshape(_NC, _NS, _NCH, _CHUNK)
    dst = jnp.concatenate(
        [edge_index[1].astype(jnp.int32),
         jnp.full((npad,), N_NODES, jnp.int32)]).reshape(
             _NC, _NS, _NCH, _CHUNK)
    b1r = b1.reshape(1, FEATURE_SIZE)
    b2r = b2.reshape(1, FP_LENGTH)
    feats = x
    fp = jnp.zeros((1, FP_LENGTH), dtype=jnp.float32)
    for _ in range(RADIUS):
        agg2 = _sc_agg(feats, src, dst)
        h, fp_part = dense(feats, agg2, W1, b1r, W2, b2r)
        fp = fp + fp_part
        feats = h
    return fp


# speedup vs baseline: 1.1227x; 1.1227x over previous
import functools

import jax
import jax.numpy as jnp
from jax.experimental import pallas as pl
from jax.experimental.pallas import tpu as pltpu
from jax.experimental.pallas import tpu_sc as plsc

FEATURE_SIZE = 128
FP_LENGTH = 512
RADIUS = 3
N_NODES = 10000
N_EDGES = 320000

_BR = 1000

_NC, _NS, _L = 2, 16, 16
_NW = _NC * _NS
_CHUNK = 128
_NCH = 80
_EPW = _NCH * _CHUNK
_EPAD = _NW * _EPW
_BCH = 16
_NB = _NCH // _BCH
_NP = 10240
_RPT = _NP // _NS
_ZC = 128
_ZCH = _RPT // _ZC


def _sc_agg_body(x_hbm, src_hbm, dst_hbm, out_hbm,
                 sidx0, didx0, sidx1, didx1, rows_a, rows_b, agg_sh,
                 sem_a, sem_b, sem_i):
    c = jax.lax.axis_index("c")
    s = jax.lax.axis_index("s")

    zero = jnp.zeros((_L,), jnp.float32)
    qpr = FEATURE_SIZE // _L

    def _zrow(i, carry):
        rows_a[i // qpr, pl.ds((i % qpr) * _L, _L)] = zero
        return carry

    jax.lax.fori_loop(0, _ZC * qpr, _zrow, 0)

    def _zcopy(k, carry):
        pltpu.sync_copy(rows_a,
                        agg_sh.at[pl.ds(s * _RPT + k * _ZC, _ZC)])
        return carry

    jax.lax.fori_loop(0, _ZCH, _zcopy, 0)
    plsc.subcore_barrier()

    ibufs = [(sidx0, didx0), (sidx1, didx1)]
    pltpu.sync_copy(src_hbm.at[c, s, pl.ds(0, _BCH)], sidx0)
    pltpu.sync_copy(dst_hbm.at[c, s, pl.ds(0, _BCH)], didx0)
    for b in range(_NB):
        sidx, didx = ibufs[b % 2]
        if b + 1 < _NB:
            nsidx, ndidx = ibufs[(b + 1) % 2]
            pltpu.async_copy(src_hbm.at[c, s, pl.ds((b + 1) * _BCH, _BCH)],
                             nsidx, sem_i)
            pltpu.async_copy(dst_hbm.at[c, s, pl.ds((b + 1) * _BCH, _BCH)],
                             ndidx, sem_i)
        pltpu.async_copy(x_hbm.at[sidx.at[0]], rows_a, sem_a)

        def _pair(i, carry):
            j0 = i * 2
            pltpu.async_copy(x_hbm.at[sidx.at[j0 + 1]], rows_b, sem_b)
            pltpu.make_async_copy(x_hbm.at[sidx.at[j0]], rows_a,
                                  sem_a).wait()
            pltpu.sync_copy(rows_a, agg_sh.at[didx.at[j0]], add=True)

            @pl.when(j0 + 2 < _BCH)
            def _():
                pltpu.async_copy(x_hbm.at[sidx.at[j0 + 2]], rows_a, sem_a)

            pltpu.make_async_copy(x_hbm.at[sidx.at[j0 + 1]], rows_b,
                                  sem_b).wait()
            pltpu.sync_copy(rows_b, agg_sh.at[didx.at[j0 + 1]], add=True)
            return carry

        jax.lax.fori_loop(0, _BCH // 2, _pair, 0)
        if b + 1 < _NB:
            nsidx, ndidx = ibufs[(b + 1) % 2]
            pltpu.make_async_copy(
                src_hbm.at[c, s, pl.ds((b + 1) * _BCH, _BCH)], nsidx,
                sem_i).wait()
            pltpu.make_async_copy(
                dst_hbm.at[c, s, pl.ds((b + 1) * _BCH, _BCH)], ndidx,
                sem_i).wait()
    plsc.subcore_barrier()

    pltpu.sync_copy(agg_sh.at[pl.ds(s * _RPT, _RPT)],
                    out_hbm.at[c, pl.ds(s * _RPT, _RPT)])


_sc_agg = pl.kernel(
    _sc_agg_body,
    out_type=jax.ShapeDtypeStruct((_NC, _NP, FEATURE_SIZE), jnp.float32),
    mesh=plsc.VectorSubcoreMesh(core_axis_name="c", subcore_axis_name="s"),
    scratch_types=[
        pltpu.VMEM((_BCH, _CHUNK), jnp.int32),
        pltpu.VMEM((_BCH, _CHUNK), jnp.int32),
        pltpu.VMEM((_BCH, _CHUNK), jnp.int32),
        pltpu.VMEM((_BCH, _CHUNK), jnp.int32),
        pltpu.VMEM((_CHUNK, FEATURE_SIZE), jnp.float32),
        pltpu.VMEM((_CHUNK, FEATURE_SIZE), jnp.float32),
        pltpu.VMEM_SHARED((_NP, FEATURE_SIZE), jnp.float32),
        pltpu.SemaphoreType.DMA,
        pltpu.SemaphoreType.DMA,
        pltpu.SemaphoreType.DMA,
    ],
)


def _dense_body(feats_ref, agg_ref, w1_ref, b1_ref, w2_ref, b2_ref,
                h_ref, fp_ref):
    i = pl.program_id(0)
    ns = feats_ref[...] + agg_ref[0] + agg_ref[1]
    h = jax.lax.dot_general(ns, w1_ref[...], (((1,), (1,)), ((), ())),
                            preferred_element_type=jnp.float32)
    h = jnp.maximum(h + b1_ref[...], 0.0)
    h_ref[...] = h
    logits = jax.lax.dot_general(h, w2_ref[...], (((1,), (1,)), ((), ())),
                                 preferred_element_type=jnp.float32)
    logits = logits + b2_ref[...]
    m = jnp.max(logits, axis=1, keepdims=True)
    e = jnp.exp(logits - m)
    p = e / jnp.sum(e, axis=1, keepdims=True)
    part = jnp.sum(p, axis=0, keepdims=True)

    @pl.when(i == 0)
    def _():
        fp_ref[...] = part

    @pl.when(i != 0)
    def _():
        fp_ref[...] = fp_ref[...] + part


def _make_dense(interpret=False):
    grid = (N_NODES // _BR,)
    return pl.pallas_call(
        _dense_body,
        grid=grid,
        in_specs=[
            pl.BlockSpec((_BR, FEATURE_SIZE), lambda i: (i, 0)),
            pl.BlockSpec((_NC, _BR, FEATURE_SIZE), lambda i: (0, i, 0)),
            pl.BlockSpec((FEATURE_SIZE, FEATURE_SIZE), lambda i: (0, 0)),
            pl.BlockSpec((1, FEATURE_SIZE), lambda i: (0, 0)),
            pl.BlockSpec((FP_LENGTH, FEATURE_SIZE), lambda i: (0, 0)),
            pl.BlockSpec((1, FP_LENGTH), lambda i: (0, 0)),
        ],
        out_specs=[
            pl.BlockSpec((_BR, FEATURE_SIZE), lambda i: (i, 0)),
            pl.BlockSpec((1, FP_LENGTH), lambda i: (0, 0)),
        ],
        out_shape=[
            jax.ShapeDtypeStruct((N_NODES, FEATURE_SIZE), jnp.float32),
            jax.ShapeDtypeStruct((1, FP_LENGTH), jnp.float32),
        ],
        interpret=interpret,
    )


def kernel(x, edge_index, W1, b1, W2, b2, interpret=False):
    dense = _make_dense(interpret)
    ppt = _EPW - N_EDGES // _NW
    src = jnp.pad(
        edge_index[0].astype(jnp.int32).reshape(_NW, N_EDGES // _NW),
        ((0, 0), (0, ppt))).reshape(_NC, _NS, _NCH, _CHUNK)
    dpad = jnp.broadcast_to(N_NODES + jnp.arange(ppt, dtype=jnp.int32),
                            (_NW, ppt))
    dst = jnp.concatenate(
        [edge_index[1].astype(jnp.int32).reshape(_NW, N_EDGES // _NW),
         dpad], axis=1).reshape(_NC, _NS, _NCH, _CHUNK)
    b1r = b1.reshape(1, FEATURE_SIZE)
    b2r = b2.reshape(1, FP_LENGTH)
    feats = x
    fp = jnp.zeros((1, FP_LENGTH), dtype=jnp.float32)
    for _ in range(RADIUS):
        agg2 = _sc_agg(feats, src, dst)
        h, fp_part = dense(feats, agg2, W1, b1r, W2, b2r)
        fp = fp + fp_part
        feats = h
    return fp


# speedup vs baseline: 2.3677x; 2.1089x over previous
import functools

import jax
import jax.numpy as jnp
from jax.experimental import pallas as pl
from jax.experimental.pallas import tpu as pltpu
from jax.experimental.pallas import tpu_sc as plsc

FEATURE_SIZE = 128
FP_LENGTH = 512
RADIUS = 3
N_NODES = 10000
N_EDGES = 320000

_BR = 1000

_NC, _NS, _L = 2, 16, 16
_NW = _NC * _NS
_EPW = N_EDGES // _NW
_CHUNK = 125
_NCH = _EPW // _CHUNK
_NP = 10240
_RPT = _NP // _NS
_ZC = 80
_ZCH = _RPT // _ZC


def _sc_agg_body(x_hbm, src_hbm, dst_hbm, out_hbm,
                 sidx_v, didx_v, rows_v, agg_sh, sem):
    c = jax.lax.axis_index("c")
    s = jax.lax.axis_index("s")

    zero = jnp.zeros((_L,), jnp.float32)
    qpr = FEATURE_SIZE // _L

    def _zrow(i, carry):
        rows_v[i // qpr, pl.ds((i % qpr) * _L, _L)] = zero
        return carry

    jax.lax.fori_loop(0, _ZC * qpr, _zrow, 0)

    def _zcopy(k, carry):
        pltpu.sync_copy(rows_v.at[pl.ds(0, _ZC)],
                        agg_sh.at[pl.ds(s * _RPT + k * _ZC, _ZC)])
        return carry

    jax.lax.fori_loop(0, _ZCH, _zcopy, 0)
    plsc.subcore_barrier()

    pltpu.sync_copy(src_hbm.at[c, s], sidx_v)
    pltpu.sync_copy(dst_hbm.at[c, s], didx_v)

    def _step(j, carry):
        pltpu.async_copy(x_hbm.at[sidx_v.at[j]], rows_v, sem).wait()
        pltpu.sync_copy(rows_v, agg_sh.at[didx_v.at[j]], add=True)
        return carry

    jax.lax.fori_loop(0, _NCH, _step, 0)
    plsc.subcore_barrier()

    pltpu.sync_copy(agg_sh.at[pl.ds(s * _RPT, _RPT)],
                    out_hbm.at[c, pl.ds(s * _RPT, _RPT)])


_sc_agg = pl.kernel(
    _sc_agg_body,
    out_type=jax.ShapeDtypeStruct((_NC, _NP, FEATURE_SIZE), jnp.float32),
    mesh=plsc.VectorSubcoreMesh(core_axis_name="c", subcore_axis_name="s"),
    scratch_types=[
        pltpu.VMEM((_NCH, _CHUNK), jnp.int32),
        pltpu.VMEM((_NCH, _CHUNK), jnp.int32),
        pltpu.VMEM((_CHUNK, FEATURE_SIZE), jnp.float32),
        pltpu.VMEM_SHARED((_NP, FEATURE_SIZE), jnp.float32),
        pltpu.SemaphoreType.DMA,
    ],
)


def _dense_body(feats_ref, agg_ref, w1_ref, b1_ref, w2_ref, b2_ref,
                h_ref, fp_ref):
    i = pl.program_id(0)
    ns = feats_ref[...] + agg_ref[0] + agg_ref[1]
    h = jax.lax.dot_general(ns, w1_ref[...], (((1,), (1,)), ((), ())),
                            preferred_element_type=jnp.float32)
    h = jnp.maximum(h + b1_ref[...], 0.0)
    h_ref[...] = h
    logits = jax.lax.dot_general(h, w2_ref[...], (((1,), (1,)), ((), ())),
                                 preferred_element_type=jnp.float32)
    logits = logits + b2_ref[...]
    m = jnp.max(logits, axis=1, keepdims=True)
    e = jnp.exp(logits - m)
    p = e / jnp.sum(e, axis=1, keepdims=True)
    part = jnp.sum(p, axis=0, keepdims=True)

    @pl.when(i == 0)
    def _():
        fp_ref[...] = part

    @pl.when(i != 0)
    def _():
        fp_ref[...] = fp_ref[...] + part


def _make_dense(interpret=False):
    grid = (N_NODES // _BR,)
    return pl.pallas_call(
        _dense_body,
        grid=grid,
        in_specs=[
            pl.BlockSpec((_BR, FEATURE_SIZE), lambda i: (i, 0)),
            pl.BlockSpec((_NC, _BR, FEATURE_SIZE), lambda i: (0, i, 0)),
            pl.BlockSpec((FEATURE_SIZE, FEATURE_SIZE), lambda i: (0, 0)),
            pl.BlockSpec((1, FEATURE_SIZE), lambda i: (0, 0)),
            pl.BlockSpec((FP_LENGTH, FEATURE_SIZE), lambda i: (0, 0)),
            pl.BlockSpec((1, FP_LENGTH), lambda i: (0, 0)),
        ],
        out_specs=[
            pl.BlockSpec((_BR, FEATURE_SIZE), lambda i: (i, 0)),
            pl.BlockSpec((1, FP_LENGTH), lambda i: (0, 0)),
        ],
        out_shape=[
            jax.ShapeDtypeStruct((N_NODES, FEATURE_SIZE), jnp.float32),
            jax.ShapeDtypeStruct((1, FP_LENGTH), jnp.float32),
        ],
        interpret=interpret,
    )


def kernel(x, edge_index, W1, b1, W2, b2, interpret=False):
    dense = _make_dense(interpret)
    src = edge_index[0].astype(jnp.int32).reshape(_NC, _NS, _NCH, _CHUNK)
    dst = edge_index[1].astype(jnp.int32).reshape(_NC, _NS, _NCH, _CHUNK)
    b1r = b1.reshape(1, FEATURE_SIZE)
    b2r = b2.reshape(1, FP_LENGTH)
    feats = x
    fp = jnp.zeros((1, FP_LENGTH), dtype=jnp.float32)
    for _ in range(RADIUS):
        agg2 = _sc_agg(feats, src, dst)
        h, fp_part = dense(feats, agg2, W1, b1r, W2, b2r)
        fp = fp + fp_part
        feats = h
    return fp


# speedup vs baseline: 3.3529x; 1.4161x over previous
import functools

import jax
import jax.numpy as jnp
from jax.experimental import pallas as pl
from jax.experimental.pallas import tpu as pltpu
from jax.experimental.pallas import tpu_sc as plsc

FEATURE_SIZE = 128
FP_LENGTH = 512
RADIUS = 3
N_NODES = 10000
N_EDGES = 320000

_BR = 1000

_NC, _NS, _L = 2, 16, 16
_NW = _NC * _NS
_EPW = N_EDGES // _NW
_CHUNK = 125
_NCH = _EPW // _CHUNK
_BCH = 16
_NB = _NCH // _BCH
_NP = 10240
_RPT = _NP // _NS
_ZC = 80
_ZCH = _RPT // _ZC


def _sc_agg_body(x_hbm, src_hbm, dst_hbm, out_hbm,
                 sidx0, didx0, sidx1, didx1, rows_a, rows_b, agg_sh,
                 sem_a, sem_b, sem_i):
    c = jax.lax.axis_index("c")
    s = jax.lax.axis_index("s")

    zero = jnp.zeros((_L,), jnp.float32)
    qpr = FEATURE_SIZE // _L

    def _zrow(i, carry):
        rows_a[i // qpr, pl.ds((i % qpr) * _L, _L)] = zero
        return carry

    jax.lax.fori_loop(0, _ZC * qpr, _zrow, 0)

    def _zcopy(k, carry):
        pltpu.sync_copy(rows_a.at[pl.ds(0, _ZC)],
                        agg_sh.at[pl.ds(s * _RPT + k * _ZC, _ZC)])
        return carry

    jax.lax.fori_loop(0, _ZCH, _zcopy, 0)
    plsc.subcore_barrier()

    ibufs = [(sidx0, didx0), (sidx1, didx1)]
    pltpu.sync_copy(src_hbm.at[c, s, pl.ds(0, _BCH)], sidx0)
    pltpu.sync_copy(dst_hbm.at[c, s, pl.ds(0, _BCH)], didx0)
    last = _BCH - 2
    for b in range(_NB):
        sidx, didx = ibufs[b % 2]
        if b + 1 < _NB:
            nsidx, ndidx = ibufs[(b + 1) % 2]
            pltpu.async_copy(src_hbm.at[c, s, pl.ds((b + 1) * _BCH, _BCH)],
                             nsidx, sem_i)
            pltpu.async_copy(dst_hbm.at[c, s, pl.ds((b + 1) * _BCH, _BCH)],
                             ndidx, sem_i)
        pltpu.async_copy(x_hbm.at[sidx.at[0]], rows_a, sem_a)

        def _pair(i, carry, sidx=sidx, didx=didx):
            j0 = i * 2
            pltpu.async_copy(x_hbm.at[sidx.at[j0 + 1]], rows_b, sem_b)
            pltpu.make_async_copy(x_hbm.at[sidx.at[j0]], rows_a,
                                  sem_a).wait()
            pltpu.sync_copy(rows_a, agg_sh.at[didx.at[j0]], add=True)
            pltpu.async_copy(x_hbm.at[sidx.at[jnp.minimum(j0 + 2, last)]],
                             rows_a, sem_a)
            pltpu.make_async_copy(x_hbm.at[sidx.at[j0 + 1]], rows_b,
                                  sem_b).wait()
            pltpu.sync_copy(rows_b, agg_sh.at[didx.at[j0 + 1]], add=True)
            return carry

        jax.lax.fori_loop(0, _BCH // 2, _pair, 0)
        pltpu.make_async_copy(x_hbm.at[sidx.at[last]], rows_a, sem_a).wait()
        if b + 1 < _NB:
            nsidx, ndidx = ibufs[(b + 1) % 2]
            pltpu.make_async_copy(
                src_hbm.at[c, s, pl.ds((b + 1) * _BCH, _BCH)], nsidx,
                sem_i).wait()
            pltpu.make_async_copy(
                dst_hbm.at[c, s, pl.ds((b + 1) * _BCH, _BCH)], ndidx,
                sem_i).wait()
    plsc.subcore_barrier()

    pltpu.sync_copy(agg_sh.at[pl.ds(s * _RPT, _RPT)],
                    out_hbm.at[c, pl.ds(s * _RPT, _RPT)])


_sc_agg = pl.kernel(
    _sc_agg_body,
    out_type=jax.ShapeDtypeStruct((_NC, _NP, FEATURE_SIZE), jnp.float32),
    mesh=plsc.VectorSubcoreMesh(core_axis_name="c", subcore_axis_name="s"),
    scratch_types=[
        pltpu.VMEM((_BCH, _CHUNK), jnp.int32),
        pltpu.VMEM((_BCH, _CHUNK), jnp.int32),
        pltpu.VMEM((_BCH, _CHUNK), jnp.int32),
        pltpu.VMEM((_BCH, _CHUNK), jnp.int32),
        pltpu.VMEM((_CHUNK, FEATURE_SIZE), jnp.float32),
        pltpu.VMEM((_CHUNK, FEATURE_SIZE), jnp.float32),
        pltpu.VMEM_SHARED((_NP, FEATURE_SIZE), jnp.float32),
        pltpu.SemaphoreType.DMA,
        pltpu.SemaphoreType.DMA,
        pltpu.SemaphoreType.DMA,
    ],
)


def _dense_body(feats_ref, agg_ref, w1_ref, b1_ref, w2_ref, b2_ref,
                h_ref, fp_ref):
    i = pl.program_id(0)
    ns = feats_ref[...] + agg_ref[0] + agg_ref[1]
    h = jax.lax.dot_general(ns, w1_ref[...], (((1,), (1,)), ((), ())),
                            preferred_element_type=jnp.float32)
    h = jnp.maximum(h + b1_ref[...], 0.0)
    h_ref[...] = h
    logits = jax.lax.dot_general(h, w2_ref[...], (((1,), (1,)), ((), ())),
                                 preferred_element_type=jnp.float32)
    logits = logits + b2_ref[...]
    m = jnp.max(logits, axis=1, keepdims=True)
    e = jnp.exp(logits - m)
    p = e / jnp.sum(e, axis=1, keepdims=True)
    part = jnp.sum(p, axis=0, keepdims=True)

    @pl.when(i == 0)
    def _():
        fp_ref[...] = part

    @pl.when(i != 0)
    def _():
        fp_ref[...] = fp_ref[...] + part


def _make_dense(interpret=False):
    grid = (N_NODES // _BR,)
    return pl.pallas_call(
        _dense_body,
        grid=grid,
        in_specs=[
            pl.BlockSpec((_BR, FEATURE_SIZE), lambda i: (i, 0)),
            pl.BlockSpec((_NC, _BR, FEATURE_SIZE), lambda i: (0, i, 0)),
            pl.BlockSpec((FEATURE_SIZE, FEATURE_SIZE), lambda i: (0, 0)),
            pl.BlockSpec((1, FEATURE_SIZE), lambda i: (0, 0)),
            pl.BlockSpec((FP_LENGTH, FEATURE_SIZE), lambda i: (0, 0)),
            pl.BlockSpec((1, FP_LENGTH), lambda i: (0, 0)),
        ],
        out_specs=[
            pl.BlockSpec((_BR, FEATURE_SIZE), lambda i: (i, 0)),
            pl.BlockSpec((1, FP_LENGTH), lambda i: (0, 0)),
        ],
        out_shape=[
            jax.ShapeDtypeStruct((N_NODES, FEATURE_SIZE), jnp.float32),
            jax.ShapeDtypeStruct((1, FP_LENGTH), jnp.float32),
        ],
        interpret=interpret,
    )


def kernel(x, edge_index, W1, b1, W2, b2, interpret=False):
    dense = _make_dense(interpret)
    src = edge_index[0].astype(jnp.int32).reshape(_NC, _NS, _NCH, _CHUNK)
    dst = edge_index[1].astype(jnp.int32).reshape(_NC, _NS, _NCH, _CHUNK)
    b1r = b1.reshape(1, FEATURE_SIZE)
    b2r = b2.reshape(1, FP_LENGTH)
    feats = x
    fp = jnp.zeros((1, FP_LENGTH), dtype=jnp.float32)
    for _ in range(RADIUS):
        agg2 = _sc_agg(feats, src, dst)
        h, fp_part = dense(feats, agg2, W1, b1r, W2, b2r)
        fp = fp + fp_part
        feats = h
    return fp


# speedup vs baseline: 3.4052x; 1.0156x over previous
import functools

import jax
import jax.numpy as jnp
from jax.experimental import pallas as pl
from jax.experimental.pallas import tpu as pltpu
from jax.experimental.pallas import tpu_sc as plsc

FEATURE_SIZE = 128
FP_LENGTH = 512
RADIUS = 3
N_NODES = 10000
N_EDGES = 320000

_BR = 1000

_NC, _NS, _L = 2, 16, 16
_NW = _NC * _NS
_EPW = N_EDGES // _NW
_CHUNK = 125
_NCH = _EPW // _CHUNK
_BCH = 16
_NB = _NCH // _BCH
_NP = 10240
_RPT = _NP // _NS
_ZC = 80
_ZCH = _RPT // _ZC


def _sc_agg_body(x_hbm, ei_hbm, out_hbm,
                 sidx0, didx0, sidx1, didx1, rows_a, rows_b, agg_sh,
                 sem_a, sem_b, sem_i):
    c = jax.lax.axis_index("c")
    s = jax.lax.axis_index("s")
    base = (c * _NS + s) * _NCH

    zero = jnp.zeros((_L,), jnp.float32)
    qpr = FEATURE_SIZE // _L

    def _zrow(i, carry):
        rows_a[i // qpr, pl.ds((i % qpr) * _L, _L)] = zero
        return carry

    jax.lax.fori_loop(0, _ZC * qpr, _zrow, 0)

    def _zcopy(k, carry):
        pltpu.sync_copy(rows_a.at[pl.ds(0, _ZC)],
                        agg_sh.at[pl.ds(s * _RPT + k * _ZC, _ZC)])
        return carry

    jax.lax.fori_loop(0, _ZCH, _zcopy, 0)
    plsc.subcore_barrier()

    ibufs = [(sidx0, didx0), (sidx1, didx1)]
    pltpu.sync_copy(ei_hbm.at[0, pl.ds(base, _BCH)], sidx0)
    pltpu.sync_copy(ei_hbm.at[1, pl.ds(base, _BCH)], didx0)
    last = _BCH - 2
    for b in range(_NB):
        sidx, didx = ibufs[b % 2]
        if b + 1 < _NB:
            nsidx, ndidx = ibufs[(b + 1) % 2]
            pltpu.async_copy(ei_hbm.at[0, pl.ds(base + (b + 1) * _BCH,
                                                _BCH)], nsidx, sem_i)
            pltpu.async_copy(ei_hbm.at[1, pl.ds(base + (b + 1) * _BCH,
                                                _BCH)], ndidx, sem_i)
        pltpu.async_copy(x_hbm.at[sidx.at[0]], rows_a, sem_a)

        def _pair(i, carry, sidx=sidx, didx=didx):
            j0 = i * 2
            pltpu.async_copy(x_hbm.at[sidx.at[j0 + 1]], rows_b, sem_b)
            pltpu.make_async_copy(x_hbm.at[sidx.at[j0]], rows_a,
                                  sem_a).wait()
            pltpu.sync_copy(rows_a, agg_sh.at[didx.at[j0]], add=True)
            pltpu.async_copy(x_hbm.at[sidx.at[jnp.minimum(j0 + 2, last)]],
                             rows_a, sem_a)
            pltpu.make_async_copy(x_hbm.at[sidx.at[j0 + 1]], rows_b,
                                  sem_b).wait()
            pltpu.sync_copy(rows_b, agg_sh.at[didx.at[j0 + 1]], add=True)
            return carry

        jax.lax.fori_loop(0, _BCH // 2, _pair, 0)
        pltpu.make_async_copy(x_hbm.at[sidx.at[last]], rows_a, sem_a).wait()
        if b + 1 < _NB:
            nsidx, ndidx = ibufs[(b + 1) % 2]
            pltpu.make_async_copy(
                ei_hbm.at[0, pl.ds(base + (b + 1) * _BCH, _BCH)], nsidx,
                sem_i).wait()
            pltpu.make_async_copy(
                ei_hbm.at[1, pl.ds(base + (b + 1) * _BCH, _BCH)], ndidx,
                sem_i).wait()
    plsc.subcore_barrier()

    pltpu.sync_copy(agg_sh.at[pl.ds(s * _RPT, _RPT)],
                    out_hbm.at[c, pl.ds(s * _RPT, _RPT)])


_sc_agg = pl.kernel(
    _sc_agg_body,
    out_type=jax.ShapeDtypeStruct((_NC, _NP, FEATURE_SIZE), jnp.float32),
    mesh=plsc.VectorSubcoreMesh(core_axis_name="c", subcore_axis_name="s"),
    scratch_types=[
        pltpu.VMEM((_BCH, _CHUNK), jnp.int32),
        pltpu.VMEM((_BCH, _CHUNK), jnp.int32),
        pltpu.VMEM((_BCH, _CHUNK), jnp.int32),
        pltpu.VMEM((_BCH, _CHUNK), jnp.int32),
        pltpu.VMEM((_CHUNK, FEATURE_SIZE), jnp.float32),
        pltpu.VMEM((_CHUNK, FEATURE_SIZE), jnp.float32),
        pltpu.VMEM_SHARED((_NP, FEATURE_SIZE), jnp.float32),
        pltpu.SemaphoreType.DMA,
        pltpu.SemaphoreType.DMA,
        pltpu.SemaphoreType.DMA,
    ],
)


def _dense_h_body(feats_ref, agg_ref, w1_ref, b1_ref, h_ref):
    ns = feats_ref[...] + agg_ref[0] + agg_ref[1]
    h = jax.lax.dot_general(ns, w1_ref[...], (((1,), (1,)), ((), ())),
                            preferred_element_type=jnp.float32)
    h_ref[...] = jnp.maximum(h + b1_ref[...], 0.0)


def _dense_fp_body(h_ref, w2_ref, b2_ref, fp_ref):
    i = pl.program_id(0)
    logits = jax.lax.dot_general(h_ref[...], w2_ref[...],
                                 (((1,), (1,)), ((), ())),
                                 preferred_element_type=jnp.float32)
    logits = logits + b2_ref[...]
    m = jnp.max(logits, axis=1, keepdims=True)
    e = jnp.exp(logits - m)
    p = e / jnp.sum(e, axis=1, keepdims=True)
    part = jnp.sum(p, axis=0, keepdims=True)

    @pl.when(i == 0)
    def _():
        fp_ref[...] = part

    @pl.when(i != 0)
    def _():
        fp_ref[...] = fp_ref[...] + part


def _make_dense_h(interpret=False):
    return pl.pallas_call(
        _dense_h_body,
        grid=(N_NODES // _BR,),
        in_specs=[
            pl.BlockSpec((_BR, FEATURE_SIZE), lambda i: (i, 0)),
            pl.BlockSpec((_NC, _BR, FEATURE_SIZE), lambda i: (0, i, 0)),
            pl.BlockSpec((FEATURE_SIZE, FEATURE_SIZE), lambda i: (0, 0)),
            pl.BlockSpec((1, FEATURE_SIZE), lambda i: (0, 0)),
        ],
        out_specs=pl.BlockSpec((_BR, FEATURE_SIZE), lambda i: (i, 0)),
        out_shape=jax.ShapeDtypeStruct((N_NODES, FEATURE_SIZE),
                                       jnp.float32),
        interpret=interpret,
    )


def _make_dense_fp(interpret=False):
    return pl.pallas_call(
        _dense_fp_body,
        grid=(N_NODES // _BR,),
        in_specs=[
            pl.BlockSpec((_BR, FEATURE_SIZE), lambda i: (i, 0)),
            pl.BlockSpec((FP_LENGTH, FEATURE_SIZE), lambda i: (0, 0)),
            pl.BlockSpec((1, FP_LENGTH), lambda i: (0, 0)),
        ],
        out_specs=pl.BlockSpec((1, FP_LENGTH), lambda i: (0, 0)),
        out_shape=jax.ShapeDtypeStruct((1, FP_LENGTH), jnp.float32),
        interpret=interpret,
    )


def kernel(x, edge_index, W1, b1, W2, b2, interpret=False):
    dense_h = _make_dense_h(interpret)
    dense_fp = _make_dense_fp(interpret)
    ei = edge_index.astype(jnp.int32).reshape(2, _NW * _NCH, _CHUNK)
    b1r = b1.reshape(1, FEATURE_SIZE)
    b2r = b2.reshape(1, FP_LENGTH)
    feats = x
    fps = []
    for _ in range(RADIUS):
        agg2 = _sc_agg(feats, ei)
        h = dense_h(feats, agg2, W1, b1r)
        fps.append(dense_fp(h, W2, b2r))
        feats = h
    return fps[0] + fps[1] + fps[2]


# speedup vs baseline: 3.4944x; 1.0262x over previous
import functools

import jax
import jax.numpy as jnp
from jax.experimental import pallas as pl
from jax.experimental.pallas import tpu as pltpu
from jax.experimental.pallas import tpu_sc as plsc

FEATURE_SIZE = 128
FP_LENGTH = 512
RADIUS = 3
N_NODES = 10000
N_EDGES = 320000

_BR = 2000

_NC, _NS, _L = 2, 16, 16
_NW = _NC * _NS
_EPW = N_EDGES // _NW
_CHUNK = 125
_NCH = _EPW // _CHUNK
_BCH = 16
_NB = _NCH // _BCH
_NP = 10240
_RPT = _NP // _NS
_ZC = 80
_ZCH = _RPT // _ZC


def _sc_agg_body(x_hbm, ei_hbm, out_hbm,
                 sidx0, didx0, sidx1, didx1, rows_a, rows_b, agg_sh,
                 sem_a, sem_b, sem_i):
    c = jax.lax.axis_index("c")
    s = jax.lax.axis_index("s")
    base = (c * _NS + s) * _NCH

    zero = jnp.zeros((_L,), jnp.float32)
    qpr = FEATURE_SIZE // _L

    def _zrow(i, carry):
        rows_a[i // qpr, pl.ds((i % qpr) * _L, _L)] = zero
        return carry

    jax.lax.fori_loop(0, _ZC * qpr, _zrow, 0)

    def _zcopy(k, carry):
        pltpu.sync_copy(rows_a.at[pl.ds(0, _ZC)],
                        agg_sh.at[pl.ds(s * _RPT + k * _ZC, _ZC)])
        return carry

    jax.lax.fori_loop(0, _ZCH, _zcopy, 0)
    plsc.subcore_barrier()

    ibufs = [(sidx0, didx0), (sidx1, didx1)]
    pltpu.sync_copy(ei_hbm.at[0, pl.ds(base, _BCH)], sidx0)
    pltpu.sync_copy(ei_hbm.at[1, pl.ds(base, _BCH)], didx0)
    last = _BCH - 2
    for b in range(_NB):
        sidx, didx = ibufs[b % 2]
        if b + 1 < _NB:
            nsidx, ndidx = ibufs[(b + 1) % 2]
            pltpu.async_copy(ei_hbm.at[0, pl.ds(base + (b + 1) * _BCH,
                                                _BCH)], nsidx, sem_i)
            pltpu.async_copy(ei_hbm.at[1, pl.ds(base + (b + 1) * _BCH,
                                                _BCH)], ndidx, sem_i)
        pltpu.async_copy(x_hbm.at[sidx.at[0]], rows_a, sem_a)

        def _pair(i, carry, sidx=sidx, didx=didx):
            j0 = i * 2
            pltpu.async_copy(x_hbm.at[sidx.at[j0 + 1]], rows_b, sem_b)
            pltpu.make_async_copy(x_hbm.at[sidx.at[j0]], rows_a,
                                  sem_a).wait()
            pltpu.sync_copy(rows_a, agg_sh.at[didx.at[j0]], add=True)
            pltpu.async_copy(x_hbm.at[sidx.at[jnp.minimum(j0 + 2, last)]],
                             rows_a, sem_a)
            pltpu.make_async_copy(x_hbm.at[sidx.at[j0 + 1]], rows_b,
                                  sem_b).wait()
            pltpu.sync_copy(rows_b, agg_sh.at[didx.at[j0 + 1]], add=True)
            return carry

        jax.lax.fori_loop(0, _BCH // 2, _pair, 0)
        pltpu.make_async_copy(x_hbm.at[sidx.at[last]], rows_a, sem_a).wait()
        if b + 1 < _NB:
            nsidx, ndidx = ibufs[(b + 1) % 2]
            pltpu.make_async_copy(
                ei_hbm.at[0, pl.ds(base + (b + 1) * _BCH, _BCH)], nsidx,
                sem_i).wait()
            pltpu.make_async_copy(
                ei_hbm.at[1, pl.ds(base + (b + 1) * _BCH, _BCH)], ndidx,
                sem_i).wait()
    plsc.subcore_barrier()

    pltpu.sync_copy(agg_sh.at[pl.ds(s * _RPT, _RPT)],
                    out_hbm.at[c, pl.ds(s * _RPT, _RPT)])


_sc_agg = pl.kernel(
    _sc_agg_body,
    out_type=jax.ShapeDtypeStruct((_NC, _NP, FEATURE_SIZE), jnp.float32),
    mesh=plsc.VectorSubcoreMesh(core_axis_name="c", subcore_axis_name="s"),
    scratch_types=[
        pltpu.VMEM((_BCH, _CHUNK), jnp.int32),
        pltpu.VMEM((_BCH, _CHUNK), jnp.int32),
        pltpu.VMEM((_BCH, _CHUNK), jnp.int32),
        pltpu.VMEM((_BCH, _CHUNK), jnp.int32),
        pltpu.VMEM((_CHUNK, FEATURE_SIZE), jnp.float32),
        pltpu.VMEM((_CHUNK, FEATURE_SIZE), jnp.float32),
        pltpu.VMEM_SHARED((_NP, FEATURE_SIZE), jnp.float32),
        pltpu.SemaphoreType.DMA,
        pltpu.SemaphoreType.DMA,
        pltpu.SemaphoreType.DMA,
    ],
)


def _dense_h_body(feats_ref, agg_ref, w1_ref, b1_ref, h_ref):
    ns = feats_ref[...] + agg_ref[0] + agg_ref[1]
    h = jax.lax.dot_general(ns, w1_ref[...], (((1,), (1,)), ((), ())),
                            preferred_element_type=jnp.float32)
    h_ref[...] = jnp.maximum(h + b1_ref[...], 0.0)


def _dense_fp_body(h_ref, w2_ref, b2_ref, fp_ref):
    i = pl.program_id(0)
    logits = jax.lax.dot_general(h_ref[...], w2_ref[...],
                                 (((1,), (1,)), ((), ())),
                                 preferred_element_type=jnp.float32)
    logits = logits + b2_ref[...]
    m = jnp.max(logits, axis=1, keepdims=True)
    e = jnp.exp(logits - m)
    p = e / jnp.sum(e, axis=1, keepdims=True)
    part = jnp.sum(p, axis=0, keepdims=True)

    @pl.when(i == 0)
    def _():
        fp_ref[...] = part

    @pl.when(i != 0)
    def _():
        fp_ref[...] = fp_ref[...] + part


def _make_dense_h(interpret=False):
    return pl.pallas_call(
        _dense_h_body,
        grid=(N_NODES // _BR,),
        in_specs=[
            pl.BlockSpec((_BR, FEATURE_SIZE), lambda i: (i, 0)),
            pl.BlockSpec((_NC, _BR, FEATURE_SIZE), lambda i: (0, i, 0)),
            pl.BlockSpec((FEATURE_SIZE, FEATURE_SIZE), lambda i: (0, 0)),
            pl.BlockSpec((1, FEATURE_SIZE), lambda i: (0, 0)),
        ],
        out_specs=pl.BlockSpec((_BR, FEATURE_SIZE), lambda i: (i, 0)),
        out_shape=jax.ShapeDtypeStruct((N_NODES, FEATURE_SIZE),
                                       jnp.float32),
        interpret=interpret,
    )


def _make_dense_fp(interpret=False):
    return pl.pallas_call(
        _dense_fp_body,
        grid=(N_NODES // _BR,),
        in_specs=[
            pl.BlockSpec((_BR, FEATURE_SIZE), lambda i: (i, 0)),
            pl.BlockSpec((FP_LENGTH, FEATURE_SIZE), lambda i: (0, 0)),
            pl.BlockSpec((1, FP_LENGTH), lambda i: (0, 0)),
        ],
        out_specs=pl.BlockSpec((1, FP_LENGTH), lambda i: (0, 0)),
        out_shape=jax.ShapeDtypeStruct((1, FP_LENGTH), jnp.float32),
        interpret=interpret,
    )


def kernel(x, edge_index, W1, b1, W2, b2, interpret=False):
    dense_h = _make_dense_h(interpret)
    dense_fp = _make_dense_fp(interpret)
    ei = edge_index.astype(jnp.int32).reshape(2, _NW * _NCH, _CHUNK)
    b1r = b1.reshape(1, FEATURE_SIZE)
    b2r = b2.reshape(1, FP_LENGTH)
    feats = x
    fps = []
    for _ in range(RADIUS):
        agg2 = _sc_agg(feats, ei)
        h = dense_h(feats, agg2, W1, b1r)
        fps.append(dense_fp(h, W2, b2r))
        feats = h
    return fps[0] + fps[1] + fps[2]


# speedup vs baseline: 3.7523x; 1.0738x over previous
import functools

import jax
import jax.numpy as jnp
from jax.experimental import pallas as pl
from jax.experimental.pallas import tpu as pltpu
from jax.experimental.pallas import tpu_sc as plsc

FEATURE_SIZE = 128
FP_LENGTH = 512
RADIUS = 3
N_NODES = 10000
N_EDGES = 320000

_BR = 2000

_NC, _NS, _L = 2, 16, 16
_NW = _NC * _NS
_EPW = N_EDGES // _NW
_CHUNK = 125
_NCH = _EPW // _CHUNK
_BCH = 16
_NB = _NCH // _BCH
_NP = 10240
_RPT = _NP // _NS
_ZC = 80
_ZCH = _RPT // _ZC


def _sc_agg_body(x_hbm, ei_hbm, out_hbm,
                 sidx0, didx0, sidx1, didx1, rows_a, rows_b, agg_sh,
                 sem_a, sem_b, sem_i):
    c = jax.lax.axis_index("c")
    s = jax.lax.axis_index("s")
    base = (c * _NS + s) * _NCH

    zero = jnp.zeros((_L,), jnp.float32)
    qpr = FEATURE_SIZE // _L

    def _zrow(i, carry):
        rows_a[i // qpr, pl.ds((i % qpr) * _L, _L)] = zero
        return carry

    jax.lax.fori_loop(0, _ZC * qpr, _zrow, 0)

    def _zcopy(k, carry):
        pltpu.sync_copy(rows_a.at[pl.ds(0, _ZC)],
                        agg_sh.at[pl.ds(s * _RPT + k * _ZC, _ZC)])
        return carry

    jax.lax.fori_loop(0, _ZCH, _zcopy, 0)
    plsc.subcore_barrier()

    ibufs = [(sidx0, didx0), (sidx1, didx1)]
    pltpu.sync_copy(ei_hbm.at[0, pl.ds(base, _BCH)], sidx0)
    pltpu.sync_copy(ei_hbm.at[1, pl.ds(base, _BCH)], didx0)
    pltpu.async_copy(x_hbm.at[sidx0.at[0]], rows_a, sem_a)
    for b in range(_NB):
        sidx, didx = ibufs[b % 2]
        if b + 1 < _NB:
            nsidx, ndidx = ibufs[(b + 1) % 2]
            pltpu.async_copy(ei_hbm.at[0, pl.ds(base + (b + 1) * _BCH,
                                                _BCH)], nsidx, sem_i)
            pltpu.async_copy(ei_hbm.at[1, pl.ds(base + (b + 1) * _BCH,
                                                _BCH)], ndidx, sem_i)

        def _pair(i, carry, sidx=sidx, didx=didx):
            j0 = i * 2
            pltpu.async_copy(x_hbm.at[sidx.at[j0 + 1]], rows_b, sem_b)
            pltpu.make_async_copy(x_hbm.at[sidx.at[j0]], rows_a,
                                  sem_a).wait()
            pltpu.sync_copy(rows_a, agg_sh.at[didx.at[j0]], add=True)
            pltpu.async_copy(x_hbm.at[sidx.at[j0 + 2]], rows_a, sem_a)
            pltpu.make_async_copy(x_hbm.at[sidx.at[j0 + 1]], rows_b,
                                  sem_b).wait()
            pltpu.sync_copy(rows_b, agg_sh.at[didx.at[j0 + 1]], add=True)
            return carry

        jax.lax.fori_loop(0, _BCH // 2 - 1, _pair, 0)
        j0 = _BCH - 2
        pltpu.async_copy(x_hbm.at[sidx.at[j0 + 1]], rows_b, sem_b)
        pltpu.make_async_copy(x_hbm.at[sidx.at[j0]], rows_a, sem_a).wait()
        pltpu.sync_copy(rows_a, agg_sh.at[didx.at[j0]], add=True)
        if b + 1 < _NB:
            nsidx, ndidx = ibufs[(b + 1) % 2]
            pltpu.make_async_copy(
                ei_hbm.at[0, pl.ds(base + (b + 1) * _BCH, _BCH)], nsidx,
                sem_i).wait()
            pltpu.make_async_copy(
                ei_hbm.at[1, pl.ds(base + (b + 1) * _BCH, _BCH)], ndidx,
                sem_i).wait()
            pltpu.async_copy(x_hbm.at[nsidx.at[0]], rows_a, sem_a)
        pltpu.make_async_copy(x_hbm.at[sidx.at[j0 + 1]], rows_b,
                              sem_b).wait()
        pltpu.sync_copy(rows_b, agg_sh.at[didx.at[j0 + 1]], add=True)
    plsc.subcore_barrier()

    pltpu.sync_copy(agg_sh.at[pl.ds(s * _RPT, _RPT)],
                    out_hbm.at[c, pl.ds(s * _RPT, _RPT)])


_sc_agg = pl.kernel(
    _sc_agg_body,
    out_type=jax.ShapeDtypeStruct((_NC, _NP, FEATURE_SIZE), jnp.float32),
    mesh=plsc.VectorSubcoreMesh(core_axis_name="c", subcore_axis_name="s"),
    scratch_types=[
        pltpu.VMEM((_BCH, _CHUNK), jnp.int32),
        pltpu.VMEM((_BCH, _CHUNK), jnp.int32),
        pltpu.VMEM((_BCH, _CHUNK), jnp.int32),
        pltpu.VMEM((_BCH, _CHUNK), jnp.int32),
        pltpu.VMEM((_CHUNK, FEATURE_SIZE), jnp.float32),
        pltpu.VMEM((_CHUNK, FEATURE_SIZE), jnp.float32),
        pltpu.VMEM_SHARED((_NP, FEATURE_SIZE), jnp.float32),
        pltpu.SemaphoreType.DMA,
        pltpu.SemaphoreType.DMA,
        pltpu.SemaphoreType.DMA,
    ],
)


def _dense_h_body(feats_ref, agg_ref, w1_ref, b1_ref, h_ref):
    ns = feats_ref[...] + agg_ref[0] + agg_ref[1]
    h = jax.lax.dot_general(ns, w1_ref[...], (((1,), (1,)), ((), ())),
                            preferred_element_type=jnp.float32)
    h_ref[...] = jnp.maximum(h + b1_ref[...], 0.0)


def _dense_fp_body(h_ref, w2_ref, b2_ref, fp_ref):
    i = pl.program_id(0)
    logits = jax.lax.dot_general(h_ref[...], w2_ref[...],
                                 (((1,), (1,)), ((), ())),
                                 preferred_element_type=jnp.float32)
    logits = logits + b2_ref[...]
    m = jnp.max(logits, axis=1, keepdims=True)
    e = jnp.exp(logits - m)
    p = e / jnp.sum(e, axis=1, keepdims=True)
    part = jnp.sum(p, axis=0, keepdims=True)

    @pl.when(i == 0)
    def _():
        fp_ref[...] = part

    @pl.when(i != 0)
    def _():
        fp_ref[...] = fp_ref[...] + part


def _make_dense_h(interpret=False):
    return pl.pallas_call(
        _dense_h_body,
        grid=(N_NODES // _BR,),
        in_specs=[
            pl.BlockSpec((_BR, FEATURE_SIZE), lambda i: (i, 0)),
            pl.BlockSpec((_NC, _BR, FEATURE_SIZE), lambda i: (0, i, 0)),
            pl.BlockSpec((FEATURE_SIZE, FEATURE_SIZE), lambda i: (0, 0)),
            pl.BlockSpec((1, FEATURE_SIZE), lambda i: (0, 0)),
        ],
        out_specs=pl.BlockSpec((_BR, FEATURE_SIZE), lambda i: (i, 0)),
        out_shape=jax.ShapeDtypeStruct((N_NODES, FEATURE_SIZE),
                                       jnp.float32),
        interpret=interpret,
    )


def _make_dense_fp(interpret=False):
    return pl.pallas_call(
        _dense_fp_body,
        grid=(N_NODES // _BR,),
        in_specs=[
            pl.BlockSpec((_BR, FEATURE_SIZE), lambda i: (i, 0)),
            pl.BlockSpec((FP_LENGTH, FEATURE_SIZE), lambda i: (0, 0)),
            pl.BlockSpec((1, FP_LENGTH), lambda i: (0, 0)),
        ],
        out_specs=pl.BlockSpec((1, FP_LENGTH), lambda i: (0, 0)),
        out_shape=jax.ShapeDtypeStruct((1, FP_LENGTH), jnp.float32),
        interpret=interpret,
    )


def kernel(x, edge_index, W1, b1, W2, b2, interpret=False):
    dense_h = _make_dense_h(interpret)
    dense_fp = _make_dense_fp(interpret)
    ei = edge_index.astype(jnp.int32).reshape(2, _NW * _NCH, _CHUNK)
    b1r = b1.reshape(1, FEATURE_SIZE)
    b2r = b2.reshape(1, FP_LENGTH)
    feats = x
    fps = []
    for _ in range(RADIUS):
        agg2 = _sc_agg(feats, ei)
        h = dense_h(feats, agg2, W1, b1r)
        fps.append(dense_fp(h, W2, b2r))
        feats = h
    return fps[0] + fps[1] + fps[2]


# speedup vs baseline: 3.7921x; 1.0106x over previous
import functools

import jax
import jax.numpy as jnp
from jax.experimental import pallas as pl
from jax.experimental.pallas import tpu as pltpu
from jax.experimental.pallas import tpu_sc as plsc

FEATURE_SIZE = 128
FP_LENGTH = 512
RADIUS = 3
N_NODES = 10000
N_EDGES = 320000

_BR = 2000

_NC, _NS, _L = 2, 16, 16
_NW = _NC * _NS
_EPW = N_EDGES // _NW
_CHUNK = 125
_NCH = _EPW // _CHUNK
_BCH = 16
_NB = _NCH // _BCH
_NP = 10240
_RPT = _NP // _NS
_ZC = 80
_ZCH = _RPT // _ZC


def _sc_agg_body(x_hbm, ei_hbm, out_hbm,
                 sidx0, didx0, sidx1, didx1, rows_a, rows_b, agg_sh,
                 sem_a, sem_b, sem_i):
    c = jax.lax.axis_index("c")
    s = jax.lax.axis_index("s")
    base = (c * _NS + s) * _NCH

    zero = jnp.zeros((_L,), jnp.float32)
    qpr = FEATURE_SIZE // _L

    def _zrow(i, carry):
        rows_a[i // qpr, pl.ds((i % qpr) * _L, _L)] = zero
        return carry

    jax.lax.fori_loop(0, _ZC * qpr, _zrow, 0)
    pltpu.async_copy(ei_hbm.at[0, pl.ds(base, _BCH)], sidx0, sem_i)
    pltpu.async_copy(ei_hbm.at[1, pl.ds(base, _BCH)], didx0, sem_i)

    def _zcopy(k, carry):
        pltpu.async_copy(rows_a.at[pl.ds(0, _ZC)],
                         agg_sh.at[pl.ds(s * _RPT + k * _ZC, _ZC)], sem_b)
        return carry

    jax.lax.fori_loop(0, _ZCH, _zcopy, 0)

    def _zwait(k, carry):
        pltpu.make_async_copy(
            rows_a.at[pl.ds(0, _ZC)],
            agg_sh.at[pl.ds(s * _RPT + k * _ZC, _ZC)], sem_b).wait()
        return carry

    jax.lax.fori_loop(0, _ZCH, _zwait, 0)
    plsc.subcore_barrier()

    ibufs = [(sidx0, didx0), (sidx1, didx1)]
    pltpu.make_async_copy(ei_hbm.at[0, pl.ds(base, _BCH)], sidx0,
                          sem_i).wait()
    pltpu.make_async_copy(ei_hbm.at[1, pl.ds(base, _BCH)], didx0,
                          sem_i).wait()
    pltpu.async_copy(x_hbm.at[sidx0.at[0]], rows_a, sem_a)
    for b in range(_NB):
        sidx, didx = ibufs[b % 2]
        if b + 1 < _NB:
            nsidx, ndidx = ibufs[(b + 1) % 2]
            pltpu.async_copy(ei_hbm.at[0, pl.ds(base + (b + 1) * _BCH,
                                                _BCH)], nsidx, sem_i)
            pltpu.async_copy(ei_hbm.at[1, pl.ds(base + (b + 1) * _BCH,
                                                _BCH)], ndidx, sem_i)

        def _pair(i, carry, sidx=sidx, didx=didx):
            j0 = i * 2
            pltpu.async_copy(x_hbm.at[sidx.at[j0 + 1]], rows_b, sem_b)
            pltpu.make_async_copy(x_hbm.at[sidx.at[j0]], rows_a,
                                  sem_a).wait()
            pltpu.sync_copy(rows_a, agg_sh.at[didx.at[j0]], add=True)
            pltpu.async_copy(x_hbm.at[sidx.at[j0 + 2]], rows_a, sem_a)
            pltpu.make_async_copy(x_hbm.at[sidx.at[j0 + 1]], rows_b,
                                  sem_b).wait()
            pltpu.sync_copy(rows_b, agg_sh.at[didx.at[j0 + 1]], add=True)
            return carry

        jax.lax.fori_loop(0, _BCH // 2 - 1, _pair, 0)
        j0 = _BCH - 2
        pltpu.async_copy(x_hbm.at[sidx.at[j0 + 1]], rows_b, sem_b)
        pltpu.make_async_copy(x_hbm.at[sidx.at[j0]], rows_a, sem_a).wait()
        pltpu.sync_copy(rows_a, agg_sh.at[didx.at[j0]], add=True)
        if b + 1 < _NB:
            nsidx, ndidx = ibufs[(b + 1) % 2]
            pltpu.make_async_copy(
                ei_hbm.at[0, pl.ds(base + (b + 1) * _BCH, _BCH)], nsidx,
                sem_i).wait()
            pltpu.make_async_copy(
                ei_hbm.at[1, pl.ds(base + (b + 1) * _BCH, _BCH)], ndidx,
                sem_i).wait()
            pltpu.async_copy(x_hbm.at[nsidx.at[0]], rows_a, sem_a)
        pltpu.make_async_copy(x_hbm.at[sidx.at[j0 + 1]], rows_b,
                              sem_b).wait()
        pltpu.sync_copy(rows_b, agg_sh.at[didx.at[j0 + 1]], add=True)
    plsc.subcore_barrier()

    pltpu.sync_copy(agg_sh.at[pl.ds(s * _RPT, _RPT)],
                    out_hbm.at[c, pl.ds(s * _RPT, _RPT)])


_sc_agg = pl.kernel(
    _sc_agg_body,
    out_type=jax.ShapeDtypeStruct((_NC, _NP, FEATURE_SIZE), jnp.float32),
    mesh=plsc.VectorSubcoreMesh(core_axis_name="c", subcore_axis_name="s"),
    scratch_types=[
        pltpu.VMEM((_BCH, _CHUNK), jnp.int32),
        pltpu.VMEM((_BCH, _CHUNK), jnp.int32),
        pltpu.VMEM((_BCH, _CHUNK), jnp.int32),
        pltpu.VMEM((_BCH, _CHUNK), jnp.int32),
        pltpu.VMEM((_CHUNK, FEATURE_SIZE), jnp.float32),
        pltpu.VMEM((_CHUNK, FEATURE_SIZE), jnp.float32),
        pltpu.VMEM_SHARED((_NP, FEATURE_SIZE), jnp.float32),
        pltpu.SemaphoreType.DMA,
        pltpu.SemaphoreType.DMA,
        pltpu.SemaphoreType.DMA,
    ],
)


def _dense_h_body(feats_ref, agg_ref, w1_ref, b1_ref, h_ref):
    ns = feats_ref[...] + agg_ref[0] + agg_ref[1]
    h = jax.lax.dot_general(ns, w1_ref[...], (((1,), (1,)), ((), ())),
                            preferred_element_type=jnp.float32)
    h_ref[...] = jnp.maximum(h + b1_ref[...], 0.0)


def _dense_fp_body(h_ref, w2_ref, b2_ref, fp_ref):
    i = pl.program_id(0)
    logits = jax.lax.dot_general(h_ref[...], w2_ref[...],
                                 (((1,), (1,)), ((), ())),
                                 preferred_element_type=jnp.float32)
    logits = logits + b2_ref[...]
    m = jnp.max(logits, axis=1, keepdims=True)
    e = jnp.exp(logits - m)
    p = e / jnp.sum(e, axis=1, keepdims=True)
    part = jnp.sum(p, axis=0, keepdims=True)

    @pl.when(i == 0)
    def _():
        fp_ref[...] = part

    @pl.when(i != 0)
    def _():
        fp_ref[...] = fp_ref[...] + part


def _make_dense_h(interpret=False):
    return pl.pallas_call(
        _dense_h_body,
        grid=(N_NODES // _BR,),
        in_specs=[
            pl.BlockSpec((_BR, FEATURE_SIZE), lambda i: (i, 0)),
            pl.BlockSpec((_NC, _BR, FEATURE_SIZE), lambda i: (0, i, 0)),
            pl.BlockSpec((FEATURE_SIZE, FEATURE_SIZE), lambda i: (0, 0)),
            pl.BlockSpec((1, FEATURE_SIZE), lambda i: (0, 0)),
        ],
        out_specs=pl.BlockSpec((_BR, FEATURE_SIZE), lambda i: (i, 0)),
        out_shape=jax.ShapeDtypeStruct((N_NODES, FEATURE_SIZE),
                                       jnp.float32),
        interpret=interpret,
    )


def _make_dense_fp(interpret=False):
    return pl.pallas_call(
        _dense_fp_body,
        grid=(N_NODES // _BR,),
        in_specs=[
            pl.BlockSpec((_BR, FEATURE_SIZE), lambda i: (i, 0)),
            pl.BlockSpec((FP_LENGTH, FEATURE_SIZE), lambda i: (0, 0)),
            pl.BlockSpec((1, FP_LENGTH), lambda i: (0, 0)),
        ],
        out_specs=pl.BlockSpec((1, FP_LENGTH), lambda i: (0, 0)),
        out_shape=jax.ShapeDtypeStruct((1, FP_LENGTH), jnp.float32),
        interpret=interpret,
    )


def kernel(x, edge_index, W1, b1, W2, b2, interpret=False):
    dense_h = _make_dense_h(interpret)
    dense_fp = _make_dense_fp(interpret)
    ei = edge_index.astype(jnp.int32).reshape(2, _NW * _NCH, _CHUNK)
    b1r = b1.reshape(1, FEATURE_SIZE)
    b2r = b2.reshape(1, FP_LENGTH)
    feats = x
    fps = []
    for _ in range(RADIUS):
        agg2 = _sc_agg(feats, ei)
        h = dense_h(feats, agg2, W1, b1r)
        fps.append(dense_fp(h, W2, b2r))
        feats = h
    return fps[0] + fps[1] + fps[2]
